# Initial kernel scaffold; baseline (speedup 1.0000x reference)
import jax
import jax.numpy as jnp
from jax.experimental import pallas as pl


def kernel(llr0, vn_adj, cn_adj, gamma):
    raise NotImplementedError("write your pallas kernel here")


# speedup vs baseline: 77.7750x; 77.7750x over previous
import functools

import jax
import jax.numpy as jnp
from jax import lax
from jax.experimental import pallas as pl
from jax.experimental.pallas import tpu as pltpu
from jax.experimental.pallas import tpu_sc as plsc

N = 100000
M = 50000
DC = 32
N_ITER = 5

NP = 100352
MP = 50048
NCHUNK = MP // 128
NW = 32
CPT = (NCHUNK + NW - 1) // NW
SLICE = NP // 16

_SIGN_BIT = jnp.int32(-2147483648)


def _sc_iter_body(v2c_hbm, adj_hbm, adjw_hbm, gamma_hbm, out_hbm,
                  v2c_v, adj_v, adjw_v, rep_v, tmp_v, gam_v, vsum_sh):
    c = lax.axis_index("c")
    s = lax.axis_index("s")
    w = s * 2 + c

    pltpu.sync_copy(v2c_hbm, v2c_v)
    pltpu.sync_copy(gamma_hbm, gam_v)

    zeros16 = jnp.zeros((16,), jnp.float32)

    def zbody(i, carry):
        tmp_v[pl.ds(i * 16, 16)] = zeros16
        return carry

    lax.fori_loop(0, SLICE // 16, zbody, 0)
    pltpu.sync_copy(tmp_v, vsum_sh.at[pl.ds(s * SLICE, SLICE)])
    plsc.subcore_barrier()

    gam = gam_v[...]
    iota = lax.iota(jnp.int32, 16)
    inf16 = jnp.full((16,), jnp.inf, jnp.float32)
    zero16i = jnp.zeros((16,), jnp.int32)

    def chunk_body(k, carry):
        chunk = w + NW * k

        @pl.when(chunk < NCHUNK)
        def _():
            pltpu.sync_copy(adj_hbm.at[chunk], adj_v)
            pltpu.sync_copy(adjw_hbm.at[chunk], adjw_v)
            row0 = chunk * 128
            for g in range(8):
                posbase = g * 512 + iota * 32

                def jbody(j, st):
                    sg, mg = st
                    pos = posbase + j
                    a = plsc.load_gather(adj_v, [pos])
                    v = plsc.load_gather(v2c_v, [a])
                    sg = lax.bitwise_xor(sg, plsc.bitcast(v, jnp.int32))
                    mg = jnp.minimum(mg, jnp.abs(v))
                    return sg, mg

                sg, mg = lax.fori_loop(0, DC, jbody, (zero16i, inf16))
                c2v = plsc.bitcast(
                    lax.bitwise_xor(plsc.bitcast(gam * mg, jnp.int32),
                                    lax.bitwise_and(sg, _SIGN_BIT)),
                    jnp.float32)
                rowglob = row0 + g * 16 + iota
                c2v = jnp.where(rowglob < M, c2v, 0.0)

                def sbody(j, carry2):
                    plsc.store_scatter(rep_v, [posbase + j], c2v)
                    return carry2

                lax.fori_loop(0, DC, sbody, 0)
            def scat_body(i, carry2):
                pltpu.sync_copy(rep_v.at[pl.ds(i * 128, 128)],
                                vsum_sh.at[adjw_v.at[i]], add=True)
                return carry2

            lax.fori_loop(0, 32, scat_body, 0)

        return carry

    lax.fori_loop(0, CPT, chunk_body, 0)
    plsc.subcore_barrier()

    pltpu.sync_copy(vsum_sh.at[pl.ds(s * SLICE, SLICE)], tmp_v)
    pltpu.sync_copy(tmp_v, out_hbm.at[c, pl.ds(s * SLICE, SLICE)])


_sc_iter = pl.kernel(
    _sc_iter_body,
    out_type=jax.ShapeDtypeStruct((2, NP), jnp.float32),
    mesh=plsc.VectorSubcoreMesh(core_axis_name="c", subcore_axis_name="s",
                                num_cores=2, num_subcores=16),
    compiler_params=pltpu.CompilerParams(needs_layout_passes=False),
    scratch_types=[
        pltpu.VMEM((NP,), jnp.float32),
        pltpu.VMEM((4096,), jnp.int32),
        pltpu.VMEM((32, 128), jnp.int32),
        pltpu.VMEM((4096,), jnp.float32),
        pltpu.VMEM((SLICE,), jnp.float32),
        pltpu.VMEM((16,), jnp.float32),
        pltpu.VMEM_SHARED((NP,), jnp.float32),
    ],
)


def _tc_update_body(llr0_ref, v2c_ref, p_ref, v2c_out, llr_out):
    vs = p_ref[0] + p_ref[1]
    nv = llr0_ref[...] + vs - v2c_ref[...]
    v2c_out[...] = nv
    llr_out[...] = llr0_ref[...] + nv


_tc_update = pl.pallas_call(
    _tc_update_body,
    out_shape=(jax.ShapeDtypeStruct((NP // 128, 128), jnp.float32),
               jax.ShapeDtypeStruct((NP // 128, 128), jnp.float32)),
)


def kernel(llr0, vn_adj, cn_adj, gamma):
    del vn_adj
    llr0_p = jnp.zeros((NP,), jnp.float32).at[:N].set(llr0)
    llr0_2d = llr0_p.reshape(NP // 128, 128)
    adj_pad = jnp.pad(cn_adj, ((0, MP - M), (0, 0)))
    adj = adj_pad.reshape(NCHUNK, 4096)
    adjw = adj_pad.reshape(NCHUNK, 32, 128)
    gamma16 = jnp.zeros((16,), jnp.float32) + gamma.astype(jnp.float32)

    v2c = jnp.zeros((NP,), jnp.float32)
    llr = llr0_p
    for _ in range(N_ITER):
        partials = _sc_iter(v2c, adj, adjw, gamma16)
        v2c_2d, llr_2d = _tc_update(
            llr0_2d, v2c.reshape(NP // 128, 128),
            partials.reshape(2, NP // 128, 128))
        v2c = v2c_2d.reshape(NP)
        llr = llr_2d.reshape(NP)
    return llr[:N]
